# baseline (speedup 1.0000x reference)
import functools

import jax
import jax.numpy as jnp
from jax import lax
from jax.experimental import pallas as pl
from jax.experimental.pallas import tpu as pltpu
from jax.experimental.pallas import tpu_sc as plsc

_NCHUNK = 4
_NSTAGE = 8
_B_SC = 12288
_TC_BLK = 512


@functools.lru_cache(maxsize=None)
def _make_gather(V, D, B, b_sc):
    info = plsc.get_sparse_core_info()
    NC, NS = info.num_cores, info.num_subcores
    NW = NC * NS
    assert b_sc % (8 * NW) == 0
    b_per_w = b_sc // NW
    C = b_per_w // _NCHUNK
    assert C * _NCHUNK == b_per_w and C % 8 == 0
    rows_per_stage = 128
    tail_start = (_NSTAGE - 1) * rows_per_stage
    tail_rows = V - tail_start
    assert 0 < tail_rows <= rows_per_stage and tail_start % 8 == 0
    mesh = plsc.VectorSubcoreMesh(core_axis_name="c", subcore_axis_name="s")

    @functools.partial(
        pl.kernel,
        mesh=mesh,
        out_type=jax.ShapeDtypeStruct((B, D), jnp.float32),
        scratch_types=[
            pltpu.VMEM_SHARED((V, D), jnp.float32),
            pltpu.VMEM((b_per_w,), jnp.int32),
            *[pltpu.VMEM((C, D), jnp.float32) for _ in range(_NCHUNK)],
            pltpu.SemaphoreType.DMA,
            pltpu.SemaphoreType.DMA,
        ],
    )
    def k(table_hbm, idx_hbm, out_hbm, table_sp, idx_v, *rest):
        bufs = rest[:_NCHUNK]
        gsem, ssem = rest[_NCHUNK:]
        cid = lax.axis_index("c")
        sid = lax.axis_index("s")
        wid = sid * NC + cid
        base = wid * b_per_w
        pltpu.sync_copy(idx_hbm.at[pl.ds(base, b_per_w)], idx_v)

        @pl.when(sid < _NSTAGE - 1)
        def _stage():
            r0 = sid * rows_per_stage
            pltpu.sync_copy(
                table_hbm.at[pl.ds(r0, rows_per_stage)],
                table_sp.at[pl.ds(r0, rows_per_stage)],
            )

        @pl.when(sid == _NSTAGE - 1)
        def _stage_tail():
            pltpu.sync_copy(
                table_hbm.at[pl.ds(tail_start, tail_rows)],
                table_sp.at[pl.ds(tail_start, tail_rows)],
            )

        plsc.subcore_barrier()
        gathers = [
            pltpu.async_copy(
                table_sp.at[idx_v.at[pl.ds(c * C, C)]], bufs[c], gsem
            )
            for c in range(_NCHUNK)
        ]
        stores = []
        for c in range(_NCHUNK):
            gathers[c].wait()
            stores.append(
                pltpu.async_copy(
                    bufs[c], out_hbm.at[pl.ds(base + c * C, C)], ssem
                )
            )
        for st in stores:
            st.wait()

    return k


@functools.lru_cache(maxsize=None)
def _make_tc_fill(V, D, B, b_sc):
    n_tc = B - b_sc
    assert n_tc % _TC_BLK == 0 and b_sc % _TC_BLK == 0
    nblk = n_tc // _TC_BLK
    off = b_sc // _TC_BLK

    def body(sc_ref, idx_ref, table_ref, out_ref):
        del sc_ref
        tv = idx_ref[...]
        oh = (
            tv[:, None]
            == lax.broadcasted_iota(jnp.int32, (_TC_BLK, V), 1)
        ).astype(jnp.float32)
        out_ref[...] = jnp.dot(
            oh, table_ref[...], preferred_element_type=jnp.float32
        )

    return pl.pallas_call(
        body,
        grid=(nblk,),
        in_specs=[
            pl.BlockSpec(memory_space=pltpu.MemorySpace.HBM),
            pl.BlockSpec((_TC_BLK,), lambda i: (i,)),
            pl.BlockSpec((V, D), lambda i: (0, 0)),
        ],
        out_specs=pl.BlockSpec((_TC_BLK, D), lambda i: (off + i, 0)),
        out_shape=jax.ShapeDtypeStruct((B, D), jnp.float32),
        input_output_aliases={0: 0},
    )


def kernel(t, time_embeddings):
    B = t.shape[0]
    V, D = time_embeddings.shape
    idx = t.reshape(B)
    sc_out = _make_gather(V, D, B, _B_SC)(time_embeddings, idx)
    return _make_tc_fill(V, D, B, _B_SC)(sc_out, idx[_B_SC:], time_embeddings)


# speedup vs baseline: 1.2250x; 1.2250x over previous
import functools

import jax
import jax.numpy as jnp
from jax import lax
from jax.experimental import pallas as pl
from jax.experimental.pallas import tpu as pltpu
from jax.experimental.pallas import tpu_sc as plsc

_CHUNKS = (64, 128, 160, 160)
_NSTAGE = 8


@functools.lru_cache(maxsize=None)
def _make_gather(V, D, B):
    info = plsc.get_sparse_core_info()
    NC, NS = info.num_cores, info.num_subcores
    NW = NC * NS
    assert B % (8 * NW) == 0
    b_per_w = B // NW
    assert sum(_CHUNKS) == b_per_w and all(c % 8 == 0 for c in _CHUNKS)
    offs = [sum(_CHUNKS[:i]) for i in range(len(_CHUNKS))]
    rows_per_stage = 128
    tail_start = (_NSTAGE - 1) * rows_per_stage
    tail_rows = V - tail_start
    assert 0 < tail_rows <= rows_per_stage and tail_start % 8 == 0
    mesh = plsc.VectorSubcoreMesh(core_axis_name="c", subcore_axis_name="s")

    @functools.partial(
        pl.kernel,
        mesh=mesh,
        out_type=jax.ShapeDtypeStruct((B, D), jnp.float32),
        scratch_types=[
            pltpu.VMEM_SHARED((V, D), jnp.float32),
            pltpu.VMEM((b_per_w,), jnp.int32),
            *[pltpu.VMEM((c, D), jnp.float32) for c in _CHUNKS],
            pltpu.SemaphoreType.DMA,
            pltpu.SemaphoreType.DMA,
        ],
    )
    def k(table_hbm, idx_hbm, out_hbm, table_sp, idx_v, *rest):
        bufs = rest[: len(_CHUNKS)]
        gsem, ssem = rest[len(_CHUNKS):]
        cid = lax.axis_index("c")
        sid = lax.axis_index("s")
        wid = sid * NC + cid
        base = wid * b_per_w
        pltpu.sync_copy(idx_hbm.at[pl.ds(base, b_per_w)], idx_v)

        @pl.when(sid < _NSTAGE - 1)
        def _stage():
            r0 = sid * rows_per_stage
            pltpu.sync_copy(
                table_hbm.at[pl.ds(r0, rows_per_stage)],
                table_sp.at[pl.ds(r0, rows_per_stage)],
            )

        @pl.when(sid == _NSTAGE - 1)
        def _stage_tail():
            pltpu.sync_copy(
                table_hbm.at[pl.ds(tail_start, tail_rows)],
                table_sp.at[pl.ds(tail_start, tail_rows)],
            )

        plsc.subcore_barrier()
        gathers = [
            pltpu.async_copy(
                table_sp.at[idx_v.at[pl.ds(offs[i], _CHUNKS[i])]],
                bufs[i],
                gsem,
            )
            for i in range(len(_CHUNKS))
        ]
        stores = []
        for i in range(len(_CHUNKS)):
            gathers[i].wait()
            stores.append(
                pltpu.async_copy(
                    bufs[i],
                    out_hbm.at[pl.ds(base + offs[i], _CHUNKS[i])],
                    ssem,
                )
            )
        for st in stores:
            st.wait()

    return k


def kernel(t, time_embeddings):
    B = t.shape[0]
    V, D = time_embeddings.shape
    idx = t.reshape(B)
    return _make_gather(V, D, B)(time_embeddings, idx)


# speedup vs baseline: 1.2332x; 1.0066x over previous
import functools

import jax
import jax.numpy as jnp
from jax import lax
from jax.experimental import pallas as pl
from jax.experimental.pallas import tpu as pltpu
from jax.experimental.pallas import tpu_sc as plsc

_CHUNKS = (128, 128, 128, 128)
_NSTAGE = 8


@functools.lru_cache(maxsize=None)
def _make_gather(V, D, B):
    info = plsc.get_sparse_core_info()
    NC, NS = info.num_cores, info.num_subcores
    NW = NC * NS
    assert B % (8 * NW) == 0
    b_per_w = B // NW
    assert sum(_CHUNKS) == b_per_w and all(c % 8 == 0 for c in _CHUNKS)
    offs = [sum(_CHUNKS[:i]) for i in range(len(_CHUNKS))]
    rows_per_stage = 128
    tail_start = (_NSTAGE - 1) * rows_per_stage
    tail_rows = V - tail_start
    assert 0 < tail_rows <= rows_per_stage and tail_start % 8 == 0
    mesh = plsc.VectorSubcoreMesh(core_axis_name="c", subcore_axis_name="s")

    @functools.partial(
        pl.kernel,
        mesh=mesh,
        out_type=jax.ShapeDtypeStruct((B, D), jnp.float32),
        scratch_types=[
            pltpu.VMEM_SHARED((V, D), jnp.float32),
            pltpu.VMEM((b_per_w,), jnp.int32),
            *[pltpu.VMEM((c, D), jnp.float32) for c in _CHUNKS],
            pltpu.SemaphoreType.DMA,
            pltpu.SemaphoreType.DMA,
        ],
    )
    def k(table_hbm, idx_hbm, out_hbm, table_sp, idx_v, *rest):
        bufs = rest[: len(_CHUNKS)]
        gsem, ssem = rest[len(_CHUNKS):]
        cid = lax.axis_index("c")
        sid = lax.axis_index("s")
        wid = sid * NC + cid
        base = wid * b_per_w
        pltpu.sync_copy(idx_hbm.at[pl.ds(base, b_per_w)], idx_v)

        @pl.when(sid < _NSTAGE - 1)
        def _stage():
            r0 = sid * rows_per_stage
            pltpu.sync_copy(
                table_hbm.at[pl.ds(r0, rows_per_stage)],
                table_sp.at[pl.ds(r0, rows_per_stage)],
            )

        @pl.when(sid == _NSTAGE - 1)
        def _stage_tail():
            pltpu.sync_copy(
                table_hbm.at[pl.ds(tail_start, tail_rows)],
                table_sp.at[pl.ds(tail_start, tail_rows)],
            )

        plsc.subcore_barrier()
        gathers = [
            pltpu.async_copy(
                table_sp.at[idx_v.at[pl.ds(offs[i], _CHUNKS[i])]],
                bufs[i],
                gsem,
            )
            for i in range(len(_CHUNKS))
        ]
        stores = []
        for i in range(len(_CHUNKS)):
            gathers[i].wait()
            stores.append(
                pltpu.async_copy(
                    bufs[i],
                    out_hbm.at[pl.ds(base + offs[i], _CHUNKS[i])],
                    ssem,
                )
            )
        for st in stores:
            st.wait()

    return k


def kernel(t, time_embeddings):
    B = t.shape[0]
    V, D = time_embeddings.shape
    idx = t.reshape(B)
    return _make_gather(V, D, B)(time_embeddings, idx)
